# baseline (speedup 1.0000x reference)
import functools

import jax
import jax.numpy as jnp
from jax import lax
from jax.experimental import pallas as pl
from jax.experimental.pallas import tpu as pltpu
from jax.experimental.pallas import tpu_sc as plsc

N = 65536
D = 256

_INFO = plsc.get_sparse_core_info()
_NC = _INFO.num_cores
_NS = _INFO.num_subcores
_NW = _NC * _NS
_CPW = N // _NW
_NV = _CPW // 16
_RADIX = 256
_NPASS = 4
_CHUNK = 128

_SC_PARAMS = pltpu.CompilerParams(use_tc_tiling_on_sc=False,
                                  needs_layout_passes=False)


def _mesh():
    return plsc.VectorSubcoreMesh(core_axis_name="c", subcore_axis_name="s")


def _wid():
    return lax.axis_index("s") * _NC + lax.axis_index("c")


def _iota16():
    return lax.iota(jnp.int32, 16)


def _srl(v, s):
    return lax.shift_right_logical(v, jnp.full((16,), s, jnp.int32))


def _take16(v, idx):
    return lax.gather(
        v, idx[:, None],
        lax.GatherDimensionNumbers(offset_dims=(), collapsed_slice_dims=(0,),
                                   start_index_map=(0,)),
        slice_sizes=(1,), mode=lax.GatherScatterMode.PROMISE_IN_BOUNDS)


def _prep_body(a_ref, l2_ref, r_ref, score_ref, mkey_ref):
    score_ref[...] = 0.5 * jnp.mean(a_ref[...], axis=1) + 0.5 * l2_ref[...]
    u = lax.bitcast_convert_type(-r_ref[...], jnp.int32)
    mkey_ref[...] = jnp.where(u < 0, ~u, u ^ jnp.int32(-2147483648))


def _prep(alignment_tensor, l2_alignment, react):
    R = 2048
    return pl.pallas_call(
        _prep_body,
        grid=(N // R,),
        in_specs=[
            pl.BlockSpec((R, D), lambda i: (i, 0)),
            pl.BlockSpec((R,), lambda i: (i,)),
            pl.BlockSpec((R,), lambda i: (i,)),
        ],
        out_specs=[pl.BlockSpec((R,), lambda i: (i,)),
                   pl.BlockSpec((R,), lambda i: (i,))],
        out_shape=[jax.ShapeDtypeStruct((N,), jnp.float32),
                   jax.ShapeDtypeStruct((N,), jnp.int32)],
    )(alignment_tensor, l2_alignment, react)


def _hist_kernel(keys, shift):
    @functools.partial(
        pl.kernel,
        mesh=_mesh(),
        out_type=jax.ShapeDtypeStruct((_NW, _RADIX), jnp.int32),
        compiler_params=_SC_PARAMS,
        scratch_types=[
            pltpu.VMEM((_CPW,), jnp.int32),
            pltpu.VMEM((16 * _RADIX,), jnp.int32),
            pltpu.VMEM((_RADIX,), jnp.int32),
        ],
    )
    def k(keys_hbm, hist_hbm, chunk_v, c2_v, row_v):
        wid = _wid()
        pltpu.sync_copy(keys_hbm.at[pl.ds(wid * _CPW, _CPW)], chunk_v)
        zero = jnp.zeros((16,), jnp.int32)
        iota = _iota16()
        ones = jnp.ones((16,), jnp.int32)
        lanebase = iota * _RADIX

        def zbody(i, c):
            c2_v[pl.ds(pl.multiple_of(i * 16, 16), 16)] = zero
            return c

        lax.fori_loop(0, 16 * _RADIX // 16, zbody, 0)

        def body(v, c):
            k16 = chunk_v[pl.ds(pl.multiple_of(v * 16, 16), 16)]
            d = _srl(k16, shift) & 255
            plsc.addupdate_scatter(c2_v, [lanebase + d], ones)
            return c

        lax.fori_loop(0, _NV, body, 0)

        def rbody(g, c):
            def sbody(l, acc):
                return acc + c2_v[pl.ds(pl.multiple_of(l * _RADIX + g * 16, 16), 16)]

            acc = lax.fori_loop(0, 16, sbody, zero)
            row_v[pl.ds(pl.multiple_of(g * 16, 16), 16)] = acc
            return c

        lax.fori_loop(0, 16, rbody, 0)
        pltpu.sync_copy(row_v, hist_hbm.at[wid])

    return k(keys)


def _scatter_kernel(keys, idxs, hist, shift):
    @functools.partial(
        pl.kernel,
        mesh=_mesh(),
        out_type=(jax.ShapeDtypeStruct((N,), jnp.int32),
                  jax.ShapeDtypeStruct((N,), jnp.int32)),
        compiler_params=_SC_PARAMS,
        scratch_types=[
            pltpu.VMEM((_NW, _RADIX), jnp.int32),
            pltpu.VMEM((_CPW,), jnp.int32),
            pltpu.VMEM((_CPW,), jnp.int32),
            pltpu.VMEM((_RADIX,), jnp.int32),
            pltpu.VMEM((_CPW,), jnp.int32),
            pltpu.VMEM((_CPW,), jnp.int32),
            pltpu.VMEM((_CPW // _CHUNK, _CHUNK), jnp.int32),
            pltpu.SemaphoreType.DMA,
        ],
    )
    def k(keys_hbm, idx_hbm, hist_hbm, ko_hbm, io_hbm,
          hist_v, kchunk, ichunk, counter_v, kout, iout, pos2, sem):
        wid = _wid()
        base = wid * _CPW
        pltpu.sync_copy(hist_hbm, hist_v)
        pltpu.sync_copy(keys_hbm.at[pl.ds(base, _CPW)], kchunk)
        pltpu.sync_copy(idx_hbm.at[pl.ds(base, _CPW)], ichunk)
        zero = jnp.zeros((16,), jnp.int32)
        iota = _iota16()

        carry = jnp.int32(0)
        for g in range(_RADIX // 16):
            def tbody(t, c):
                accg, myg = c
                h = hist_v[t, pl.ds(g * 16, 16)]
                myg = jnp.where(t == wid, accg, myg)
                return accg + h, myg

            accg, myg = lax.fori_loop(0, _NW, tbody, (zero, zero))
            cs = plsc.cumsum(accg)
            offg = (cs - accg) + myg + jnp.broadcast_to(carry, (16,))
            counter_v[pl.ds(g * 16, 16)] = offg
            carry = carry + jnp.sum(accg)

        def body(v, c):
            off = pl.ds(pl.multiple_of(v * 16, 16), 16)
            k16 = kchunk[off]
            i16 = ichunk[off]
            d = _srl(k16, shift) & 255
            comp = d * 16 + iota
            sk = plsc.sort_key_val(comp, comp)
            if isinstance(sk, (tuple, list)):
                sk = sk[0]
            sd = _srl(sk, 4)
            sl = sk & 15
            prev = _take16(sd, jnp.maximum(iota - 1, 0))
            first = (sd != prev) | (iota == 0)
            startpos = plsc.cummax(jnp.where(first, iota, 0))
            rank = iota - startpos
            bases = plsc.load_gather(counter_v, [sd])
            pos = bases + rank
            firsti = jnp.where(first, 1, 0).astype(jnp.int32)
            nf = _take16(firsti, jnp.minimum(iota + 1, 15))
            last = (iota == 15) | (nf == 1)
            plsc.store_scatter(counter_v, [sd], pos + 1, mask=last)
            kout[off] = _take16(k16, sl)
            iout[off] = _take16(i16, sl)
            row = lax.shift_right_logical(v, 3)
            col = (v & 7) * 16
            pos2[row, pl.ds(pl.multiple_of(col, 16), 16)] = pos
            return c

        lax.fori_loop(0, _NV, body, 0)

        descs = []
        for j in range(_CPW // _CHUNK):
            sl_ = pl.ds(j * _CHUNK, _CHUNK)
            descs.append(pltpu.async_copy(kout.at[sl_], ko_hbm.at[pos2.at[j]], sem))
            descs.append(pltpu.async_copy(iout.at[sl_], io_hbm.at[pos2.at[j]], sem))
        for dd in descs:
            dd.wait()

    return k(keys, idxs, hist)


def _sc_sort(mkeys):
    keys = mkeys
    idxs = jnp.arange(N, dtype=jnp.int32)
    for p in range(_NPASS):
        shift = 8 * p
        hist = _hist_kernel(keys, shift)
        keys, idxs = _scatter_kernel(keys, idxs, hist, shift)
    return idxs


def _gather_rows(table, idx2):
    b_per_w = N // _NW
    n_chunks = b_per_w // _CHUNK

    @functools.partial(
        pl.kernel,
        mesh=_mesh(),
        out_type=jax.ShapeDtypeStruct((N, 16), jnp.float32),
        compiler_params=_SC_PARAMS,
        scratch_types=[
            pltpu.VMEM((n_chunks, _CHUNK), jnp.int32),
            pltpu.VMEM((b_per_w, 16), jnp.float32),
            pltpu.SemaphoreType.DMA,
        ],
    )
    def k(table_hbm, idx_hbm, out_hbm, idx_v, rows_v, sem):
        wid = _wid()
        base = wid * b_per_w
        pltpu.sync_copy(idx_hbm.at[pl.ds(wid * n_chunks, n_chunks)], idx_v)
        descs = []
        for j in range(n_chunks):
            descs.append(pltpu.async_copy(
                table_hbm.at[idx_v.at[j]],
                rows_v.at[pl.ds(j * _CHUNK, _CHUNK)],
                sem,
            ))
        for dd in descs:
            dd.wait()
        pltpu.sync_copy(rows_v, out_hbm.at[pl.ds(base, b_per_w)])

    return k(table, idx2)


def kernel(alignment_tensor, l2_alignment, effective_reactivity, atom_indices,
           refined_peak_points, refined_peak_values, approach_vectors,
           exposure_scores):
    alignment_score, mkeys = _prep(alignment_tensor, l2_alignment,
                                   effective_reactivity)

    order = _sc_sort(mkeys)

    packed = jnp.concatenate([
        lax.bitcast_convert_type(atom_indices, jnp.float32)[:, None],
        refined_peak_points,
        refined_peak_values[:, None],
        approach_vectors,
        alignment_score[:, None],
        exposure_scores[:, None],
        effective_reactivity[:, None],
        jnp.zeros((N, 5), jnp.float32),
    ], axis=1)

    rows = _gather_rows(packed, order.reshape(N // _CHUNK, _CHUNK))

    ranked_atom_indices = lax.bitcast_convert_type(rows[:, 0], jnp.int32)
    som_coordinates = rows[:, 1:4]
    psi_peak = rows[:, 4]
    approach_vector = rows[:, 5:8]
    alignment_score_ranked = rows[:, 8]
    exposure_score = rows[:, 9]
    effective_reactivity_ranked = rows[:, 10]
    return (ranked_atom_indices, som_coordinates, psi_peak, approach_vector,
            alignment_score_ranked, exposure_score, effective_reactivity_ranked)


# speedup vs baseline: 2.6269x; 2.6269x over previous
import functools

import jax
import jax.numpy as jnp
from jax import lax
from jax.experimental import pallas as pl
from jax.experimental.pallas import tpu as pltpu
from jax.experimental.pallas import tpu_sc as plsc

N = 65536
D = 256

_INFO = plsc.get_sparse_core_info()
_NC = _INFO.num_cores
_NS = _INFO.num_subcores
_NW = _NC * _NS
_CPW = N // _NW
_NV = _CPW // 16
_SHARE = N // _NS
_RADIX = 256
_NPASS = 4
_CHUNK = 128

_SC_PARAMS = pltpu.CompilerParams(use_tc_tiling_on_sc=False,
                                  needs_layout_passes=False)


def _mesh():
    return plsc.VectorSubcoreMesh(core_axis_name="c", subcore_axis_name="s")


def _wid():
    return lax.axis_index("s") * _NC + lax.axis_index("c")


def _iota16():
    return lax.iota(jnp.int32, 16)


def _srl(v, s):
    return lax.shift_right_logical(v, jnp.full((16,), s, jnp.int32))


def _prep_body(a_ref, l2_ref, r_ref, score_ref, mkey_ref):
    score_ref[...] = 0.5 * jnp.mean(a_ref[...], axis=1) + 0.5 * l2_ref[...]
    u = lax.bitcast_convert_type(-r_ref[...], jnp.int32)
    mkey_ref[...] = jnp.where(u < 0, ~u, u ^ jnp.int32(-2147483648))


def _prep(alignment_tensor, l2_alignment, react):
    R = 2048
    return pl.pallas_call(
        _prep_body,
        grid=(N // R,),
        in_specs=[
            pl.BlockSpec((R, D), lambda i: (i, 0)),
            pl.BlockSpec((R,), lambda i: (i,)),
            pl.BlockSpec((R,), lambda i: (i,)),
        ],
        out_specs=[pl.BlockSpec((R,), lambda i: (i,)),
                   pl.BlockSpec((R,), lambda i: (i,))],
        out_shape=[jax.ShapeDtypeStruct((N,), jnp.float32),
                   jax.ShapeDtypeStruct((N,), jnp.int32)],
    )(alignment_tensor, l2_alignment, react)


def _merge16(kc0, kc1, ic1, off):
    i1 = ic1[off]
    sel = i1 >= 0
    return jnp.where(sel, kc1[off], kc0[off]), sel, i1


def _hist_kernel(k2, i2, shift):
    @functools.partial(
        pl.kernel,
        mesh=_mesh(),
        out_type=jax.ShapeDtypeStruct((_NW, _RADIX), jnp.int32),
        compiler_params=_SC_PARAMS,
        scratch_types=[
            pltpu.VMEM((_CPW,), jnp.int32),
            pltpu.VMEM((_CPW,), jnp.int32),
            pltpu.VMEM((_CPW,), jnp.int32),
            pltpu.VMEM((16 * _RADIX,), jnp.int32),
            pltpu.VMEM((_RADIX,), jnp.int32),
        ],
    )
    def k(k_hbm, i_hbm, hist_hbm, kc0, kc1, ic1, c2_v, row_v):
        wid = _wid()
        sl = pl.ds(wid * _CPW, _CPW)
        pltpu.sync_copy(k_hbm.at[0, sl], kc0)
        pltpu.sync_copy(k_hbm.at[1, sl], kc1)
        pltpu.sync_copy(i_hbm.at[1, sl], ic1)
        zero = jnp.zeros((16,), jnp.int32)
        iota = _iota16()
        ones = jnp.ones((16,), jnp.int32)
        lanebase = iota * _RADIX

        def zbody(i, c):
            c2_v[pl.ds(pl.multiple_of(i * 16, 16), 16)] = zero
            return c

        lax.fori_loop(0, 16 * _RADIX // 16, zbody, 0)

        def body(v, c):
            off = pl.ds(pl.multiple_of(v * 16, 16), 16)
            k16, _, _ = _merge16(kc0, kc1, ic1, off)
            d = _srl(k16, shift) & 255
            plsc.addupdate_scatter(c2_v, [lanebase + d], ones)
            return c

        lax.fori_loop(0, _NV, body, 0)

        def rbody(g, c):
            def sbody(l, acc):
                return acc + c2_v[pl.ds(pl.multiple_of(l * _RADIX + g * 16, 16), 16)]

            acc = lax.fori_loop(0, 16, sbody, zero)
            row_v[pl.ds(pl.multiple_of(g * 16, 16), 16)] = acc
            return c

        lax.fori_loop(0, 16, rbody, 0)
        pltpu.sync_copy(row_v, hist_hbm.at[wid])

    return k(k2, i2)


def _scatter_kernel(k2, i2, hist, shift):
    @functools.partial(
        pl.kernel,
        mesh=_mesh(),
        out_type=(jax.ShapeDtypeStruct((_NC, N), jnp.int32),
                  jax.ShapeDtypeStruct((_NC, N), jnp.int32)),
        compiler_params=_SC_PARAMS,
        scratch_types=[
            pltpu.VMEM((_NW, _RADIX), jnp.int32),
            pltpu.VMEM((_CPW,), jnp.int32),
            pltpu.VMEM((_CPW,), jnp.int32),
            pltpu.VMEM((_CPW,), jnp.int32),
            pltpu.VMEM((_CPW,), jnp.int32),
            pltpu.VMEM((_RADIX,), jnp.int32),
            pltpu.VMEM((_CPW,), jnp.int32),
            pltpu.VMEM((_CPW,), jnp.int32),
            pltpu.VMEM((_CPW // _CHUNK, _CHUNK), jnp.int32),
            pltpu.VMEM((_SHARE,), jnp.int32),
            pltpu.VMEM_SHARED((N,), jnp.int32),
            pltpu.VMEM_SHARED((N,), jnp.int32),
            pltpu.SemaphoreType.DMA,
        ],
    )
    def k(k_hbm, i_hbm, hist_hbm, ko_hbm, io_hbm,
          hist_v, kc0, kc1, ic0, ic1, counter_v, kmerged, imerged, pos2,
          neg_v, kshared, ishared, sem):
        cc = lax.axis_index("c")
        ss = lax.axis_index("s")
        wid = ss * _NC + cc
        sl = pl.ds(wid * _CPW, _CPW)
        pltpu.sync_copy(hist_hbm, hist_v)
        pltpu.sync_copy(k_hbm.at[0, sl], kc0)
        pltpu.sync_copy(k_hbm.at[1, sl], kc1)
        pltpu.sync_copy(i_hbm.at[0, sl], ic0)
        pltpu.sync_copy(i_hbm.at[1, sl], ic1)
        zero = jnp.zeros((16,), jnp.int32)
        iota = _iota16()
        negones = jnp.full((16,), -1, jnp.int32)

        def nbody(i, c):
            neg_v[pl.ds(pl.multiple_of(i * 16, 16), 16)] = negones
            return c

        lax.fori_loop(0, _SHARE // 16, nbody, 0)
        my_slice = pl.ds(ss * _SHARE, _SHARE)
        pltpu.sync_copy(neg_v, ishared.at[my_slice])

        carry = jnp.int32(0)
        for g in range(_RADIX // 16):
            def tbody(t, c):
                accg, myg = c
                h = hist_v[t, pl.ds(g * 16, 16)]
                myg = jnp.where(t == wid, accg, myg)
                return accg + h, myg

            accg, myg = lax.fori_loop(0, _NW, tbody, (zero, zero))
            cs = plsc.cumsum(accg)
            offg = (cs - accg) + myg + jnp.broadcast_to(carry, (16,))
            counter_v[pl.ds(g * 16, 16)] = offg
            carry = carry + jnp.sum(accg)

        plsc.subcore_barrier()

        def body(v, c):
            off = pl.ds(pl.multiple_of(v * 16, 16), 16)
            k16, sel, i1 = _merge16(kc0, kc1, ic1, off)
            i16 = jnp.where(sel, i1, ic0[off])
            d = _srl(k16, shift) & 255
            cnt, lastm = plsc.scan_count(d)
            bases = plsc.load_gather(counter_v, [d])
            pos = bases + cnt - 1
            plsc.store_scatter(counter_v, [d], pos + 1, mask=lastm)
            kmerged[off] = k16
            imerged[off] = i16
            row = lax.shift_right_logical(v, 3)
            col = (v & 7) * 16
            pos2[row, pl.ds(pl.multiple_of(col, 16), 16)] = pos
            return c

        lax.fori_loop(0, _NV, body, 0)

        descs = []
        for j in range(_CPW // _CHUNK):
            sj = pl.ds(j * _CHUNK, _CHUNK)
            descs.append(pltpu.async_copy(kmerged.at[sj], kshared.at[pos2.at[j]], sem))
            descs.append(pltpu.async_copy(imerged.at[sj], ishared.at[pos2.at[j]], sem))
        for dd in descs:
            dd.wait()
        plsc.subcore_barrier()

        pltpu.sync_copy(kshared.at[my_slice], ko_hbm.at[cc, my_slice])
        pltpu.sync_copy(ishared.at[my_slice], io_hbm.at[cc, my_slice])

    return k(k2, i2, hist)


def _sc_sort(mkeys):
    k2 = jnp.stack([mkeys, mkeys])
    i2 = jnp.stack([jnp.arange(N, dtype=jnp.int32),
                    jnp.full((N,), -1, jnp.int32)])
    for p in range(_NPASS):
        hist = _hist_kernel(k2, i2, 8 * p)
        k2, i2 = _scatter_kernel(k2, i2, hist, 8 * p)
    return i2


def _gather_rows(table, idx3):
    b_per_w = N // _NW
    n_chunks = b_per_w // _CHUNK

    @functools.partial(
        pl.kernel,
        mesh=_mesh(),
        out_type=jax.ShapeDtypeStruct((N, 16), jnp.float32),
        compiler_params=_SC_PARAMS,
        scratch_types=[
            pltpu.VMEM((n_chunks, _CHUNK), jnp.int32),
            pltpu.VMEM((n_chunks, _CHUNK), jnp.int32),
            pltpu.VMEM((n_chunks, _CHUNK), jnp.int32),
            pltpu.VMEM((b_per_w, 16), jnp.float32),
            pltpu.SemaphoreType.DMA,
        ],
    )
    def k(table_hbm, idx_hbm, out_hbm, i0_v, i1_v, idx_v, rows_v, sem):
        wid = _wid()
        base = wid * b_per_w
        rsl = pl.ds(wid * n_chunks, n_chunks)
        pltpu.sync_copy(idx_hbm.at[0, rsl], i0_v)
        pltpu.sync_copy(idx_hbm.at[1, rsl], i1_v)

        def mbody(v, c):
            row = lax.shift_right_logical(v, 3)
            col = (v & 7) * 16
            csl = pl.ds(pl.multiple_of(col, 16), 16)
            a = i0_v[row, csl]
            b = i1_v[row, csl]
            idx_v[row, csl] = jnp.where(b >= 0, b, a)
            return c

        lax.fori_loop(0, n_chunks * 8, mbody, 0)

        descs = []
        for j in range(n_chunks):
            descs.append(pltpu.async_copy(
                table_hbm.at[idx_v.at[j]],
                rows_v.at[pl.ds(j * _CHUNK, _CHUNK)],
                sem,
            ))
        for dd in descs:
            dd.wait()
        pltpu.sync_copy(rows_v, out_hbm.at[pl.ds(base, b_per_w)])

    return k(table, idx3)


def kernel(alignment_tensor, l2_alignment, effective_reactivity, atom_indices,
           refined_peak_points, refined_peak_values, approach_vectors,
           exposure_scores):
    alignment_score, mkeys = _prep(alignment_tensor, l2_alignment,
                                   effective_reactivity)

    i2 = _sc_sort(mkeys)

    packed = jnp.concatenate([
        lax.bitcast_convert_type(atom_indices, jnp.float32)[:, None],
        refined_peak_points,
        refined_peak_values[:, None],
        approach_vectors,
        alignment_score[:, None],
        exposure_scores[:, None],
        effective_reactivity[:, None],
        jnp.zeros((N, 5), jnp.float32),
    ], axis=1)

    rows = _gather_rows(packed, i2.reshape(2, N // _CHUNK, _CHUNK))

    ranked_atom_indices = lax.bitcast_convert_type(rows[:, 0], jnp.int32)
    som_coordinates = rows[:, 1:4]
    psi_peak = rows[:, 4]
    approach_vector = rows[:, 5:8]
    alignment_score_ranked = rows[:, 8]
    exposure_score = rows[:, 9]
    effective_reactivity_ranked = rows[:, 10]
    return (ranked_atom_indices, som_coordinates, psi_peak, approach_vector,
            alignment_score_ranked, exposure_score, effective_reactivity_ranked)
